# Initial kernel scaffold; baseline (speedup 1.0000x reference)
import jax
import jax.numpy as jnp
from jax.experimental import pallas as pl


def kernel(inputs, gcn_pred, gcn_labels, train_mask, tr_mask, tcl_mask, radii_map, sin_map, cos_map):
    raise NotImplementedError("write your pallas kernel here")


# speedup vs baseline: 13.4148x; 13.4148x over previous
import jax
import jax.numpy as jnp
from jax.experimental import pallas as pl
from jax.experimental.pallas import tpu as pltpu

LANES = 128


def _sl1(r):
    d = jnp.abs(r - 1.0)
    return jnp.where(d < 1.0, 0.5 * d * d, d - 0.5)


def _sl1_custom(x, t):
    d = jnp.abs(x - t)
    return jnp.where(d < 1.0 / 9.0, 4.5 * d * d, d - 1.0 / 18.0)


def _ce2(a, b, label):
    m = jnp.maximum(a, b)
    sp = jnp.log(1.0 + jnp.exp(-jnp.abs(a - b)))
    chosen = jnp.where(label == 1, b, a)
    return m - chosen + sp


def _make_body(nblk, rb, rows_total, g_rows):
    last_b = None

    def body(in_ref, tr_ref, tn_ref, tcl_ref, rad_ref, sin_ref, cos_ref,
             gp_ref, gl_ref, out_ref, acc_ref, ce_ref):
        b = pl.program_id(0)
        j = pl.program_id(1)
        nb = pl.num_programs(0)
        step = b * nblk + j

        @pl.when(step == 0)
        def _init():
            acc_ref[...] = jnp.zeros_like(acc_ref)

        trm = tr_ref[0]
        tnm = tn_ref[0]
        tclm = tcl_ref[0]
        posf = (trm * tnm).astype(jnp.float32)
        negf = ((1 - trm) * tnm).astype(jnp.float32)

        l0 = in_ref[0, 0]
        l1 = in_ref[0, 1]
        ce_tr = _ce2(l0, l1, trm)

        t0 = in_ref[0, 2]
        t1 = in_ref[0, 3]
        ce_tcl = _ce2(t0, t1, tclm)

        sn = in_ref[0, 4]
        cs = in_ref[0, 5]
        scale = jax.lax.rsqrt(sn * sn + cs * cs + 0.0001)
        snp = sn * scale
        csp = cs * scale

        tp = in_ref[0, 6]
        bt = in_ref[0, 7]
        topm = rad_ref[0, 0]
        botm = rad_ref[0, 1]
        rad_l = _sl1(tp / (topm + 0.01)) + _sl1(bt / (botm + 0.01))

        tcl_sel = tclm == 1
        s0 = posf
        s1 = negf
        s2 = posf * ce_tr
        s3 = negf * ce_tr
        s4 = posf * ce_tcl
        s5 = tclm.astype(jnp.float32)
        s6 = (tnm * tclm).astype(jnp.float32)
        s7 = jnp.where(tcl_sel, rad_l, 0.0)
        s8 = jnp.where(tcl_sel, _sl1_custom(snp, sin_ref[0]), 0.0)
        s9 = jnp.where(tcl_sel, _sl1_custom(csp, cos_ref[0]), 0.0)

        part = jnp.stack(
            [jnp.sum(s, axis=0) for s in (s0, s1, s2, s3, s4, s5, s6, s7, s8, s9)],
            axis=0)
        acc_ref[0:10, :] += part

        ce_ref[pl.ds(step * rb, rb), :] = jnp.where(negf > 0.0, ce_tr, -1.0)

        @pl.when(step == nb * nblk - 1)
        def _fin():
            n_pos = jnp.sum(acc_ref[0, :])
            n_neg_avail = jnp.sum(acc_ref[1, :])
            loss_pos = jnp.sum(acc_ref[2, :])
            neg_sum = jnp.sum(acc_ref[3, :])
            tcl_sum = jnp.sum(acc_ref[4, :])
            n_tcl_sel = jnp.sum(acc_ref[5, :])
            n_tcl_train = jnp.sum(acc_ref[6, :])
            radii_sum = jnp.sum(acc_ref[7, :])
            sin_sum = jnp.sum(acc_ref[8, :])
            cos_sum = jnp.sum(acc_ref[9, :])

            n_neg = jnp.where(
                n_pos > 0.0,
                jnp.minimum(n_neg_avail, jnp.floor(3.0 * n_pos)),
                jnp.float32(100.0))

            cr = min(512, rows_total)
            nch = rows_total // cr

            def _count_ge(cand):
                def it(c, acc):
                    blk = ce_ref[pl.ds(c * cr, cr), :]
                    bits = jax.lax.bitcast_convert_type(blk, jnp.int32)
                    return acc + jnp.sum((bits >= cand).astype(jnp.float32))
                return jax.lax.fori_loop(0, nch, it, jnp.float32(0.0))

            def _topk_rare(_):
                k_f = n_neg

                def bit_it(i, t):
                    cand = t | (jnp.int32(1) << (30 - i))
                    cnt = _count_ge(cand)
                    return jnp.where(cnt >= k_f, cand, t)

                t = jax.lax.fori_loop(0, 31, bit_it, jnp.int32(0))

                def fin_it(c, carry):
                    s, n = carry
                    blk = ce_ref[pl.ds(c * cr, cr), :]
                    bits = jax.lax.bitcast_convert_type(blk, jnp.int32)
                    gt = bits > t
                    s = s + jnp.sum(jnp.where(gt, blk, 0.0))
                    n = n + jnp.sum(gt.astype(jnp.float32))
                    return (s, n)

                s_gt, n_gt = jax.lax.fori_loop(
                    0, nch, fin_it, (jnp.float32(0.0), jnp.float32(0.0)))
                t_val = jax.lax.bitcast_convert_type(t, jnp.float32)
                return s_gt + (k_f - n_gt) * t_val

            topk_sum = jax.lax.cond(
                n_neg >= n_neg_avail, lambda _: neg_sum, _topk_rare, 0)

            loss_tr = (loss_pos + topk_sum) / (n_pos + n_neg)
            loss_tcl = jnp.where(
                n_pos > 0.0, tcl_sum / jnp.maximum(n_pos, 1.0), 0.0)
            denom = jnp.maximum(n_tcl_sel, 1.0)
            cond = n_tcl_train > 0.0
            loss_radii = jnp.where(cond, radii_sum / denom, 0.0)
            loss_sin = jnp.where(cond, sin_sum / denom, 0.0)
            loss_cos = jnp.where(cond, cos_sum / denom, 0.0)

            gce = _ce2(gp_ref[0], gp_ref[1], gl_ref[...])
            gcn_loss = jnp.sum(gce) / jnp.float32(g_rows * LANES)

            lane = jax.lax.broadcasted_iota(jnp.int32, (1, LANES), 1)
            outv = (jnp.where(lane == 0, loss_tr, 0.0)
                    + jnp.where(lane == 1, loss_tcl, 0.0)
                    + jnp.where(lane == 2, loss_sin, 0.0)
                    + jnp.where(lane == 3, loss_cos, 0.0)
                    + jnp.where(lane == 4, loss_radii, 0.0)
                    + jnp.where(lane == 5, gcn_loss, 0.0))
            out_ref[...] = outv

    return body


def _run(inputs, gcn_pred, gcn_labels, train_mask, tr_mask, tcl_mask,
         radii_map, sin_map, cos_map, interpret=False):
    b, c, h, w = inputs.shape
    pix = h * w
    rb = 128
    nblk = pix // (rb * LANES)
    rows_b = pix // LANES
    rows_total = b * rows_b
    g = gcn_pred.shape[0]
    g_rows = g // LANES

    x = inputs.reshape(b, 8, rows_b, LANES)
    trm = tr_mask.astype(jnp.int32).reshape(b, rows_b, LANES)
    tnm = train_mask.astype(jnp.int32).reshape(b, rows_b, LANES)
    tcl = tcl_mask.astype(jnp.int32).reshape(b, rows_b, LANES)
    rad = jnp.transpose(radii_map.reshape(b, pix, 2), (0, 2, 1))
    rad = rad.reshape(b, 2, rows_b, LANES)
    snm = sin_map.reshape(b, rows_b, LANES)
    csm = cos_map.reshape(b, rows_b, LANES)
    gp = jnp.transpose(gcn_pred, (1, 0)).reshape(2, g_rows, LANES)
    gl = gcn_labels.astype(jnp.int32).reshape(g_rows, LANES)

    body = _make_body(nblk, rb, rows_total, g_rows)
    out = pl.pallas_call(
        body,
        grid=(b, nblk),
        in_specs=[
            pl.BlockSpec((1, 8, rb, LANES), lambda i, j: (i, 0, j, 0)),
            pl.BlockSpec((1, rb, LANES), lambda i, j: (i, j, 0)),
            pl.BlockSpec((1, rb, LANES), lambda i, j: (i, j, 0)),
            pl.BlockSpec((1, rb, LANES), lambda i, j: (i, j, 0)),
            pl.BlockSpec((1, 2, rb, LANES), lambda i, j: (i, 0, j, 0)),
            pl.BlockSpec((1, rb, LANES), lambda i, j: (i, j, 0)),
            pl.BlockSpec((1, rb, LANES), lambda i, j: (i, j, 0)),
            pl.BlockSpec((2, g_rows, LANES), lambda i, j: (0, 0, 0)),
            pl.BlockSpec((g_rows, LANES), lambda i, j: (0, 0)),
        ],
        out_specs=pl.BlockSpec((1, LANES), lambda i, j: (0, 0)),
        out_shape=jax.ShapeDtypeStruct((1, LANES), jnp.float32),
        scratch_shapes=[
            pltpu.VMEM((16, LANES), jnp.float32),
            pltpu.VMEM((rows_total, LANES), jnp.float32),
        ],
        interpret=interpret,
    )(x, trm, tnm, tcl, rad, snm, csm, gp, gl)
    return (out[0, 0], out[0, 1], out[0, 2], out[0, 3], out[0, 4], out[0, 5])


def kernel(inputs, gcn_pred, gcn_labels, train_mask, tr_mask, tcl_mask,
           radii_map, sin_map, cos_map):
    return _run(inputs, gcn_pred, gcn_labels, train_mask, tr_mask, tcl_mask,
                radii_map, sin_map, cos_map)


# speedup vs baseline: 16.5333x; 1.2325x over previous
import jax
import jax.numpy as jnp
from jax.experimental import pallas as pl
from jax.experimental.pallas import tpu as pltpu

LANES = 128


def _sl1(r):
    d = jnp.abs(r - 1.0)
    return jnp.where(d < 1.0, 0.5 * d * d, d - 0.5)


def _sl1_custom(x, t):
    d = jnp.abs(x - t)
    return jnp.where(d < 1.0 / 9.0, 4.5 * d * d, d - 1.0 / 18.0)


def _ce2(a, b, label):
    m = jnp.maximum(a, b)
    sp = jnp.log(1.0 + jnp.exp(-jnp.abs(a - b)))
    chosen = jnp.where(label == 1, b, a)
    return m - chosen + sp


def _make_body(nblk, rb, rows_total, g_rows):
    last_b = None

    def body(in_ref, tr_ref, tn_ref, tcl_ref, rad_ref, sin_ref, cos_ref,
             gp_ref, gl_ref, out_ref, acc_ref, ce_ref):
        b = pl.program_id(0)
        j = pl.program_id(1)
        nb = pl.num_programs(0)
        step = b * nblk + j

        @pl.when(step == 0)
        def _init():
            acc_ref[...] = jnp.zeros_like(acc_ref)

        trm = tr_ref[0]
        tnm = tn_ref[0]
        tclm = tcl_ref[0]
        posf = (trm * tnm).astype(jnp.float32)
        negf = ((1 - trm) * tnm).astype(jnp.float32)

        l0 = in_ref[0, 0]
        l1 = in_ref[0, 1]
        ce_tr = _ce2(l0, l1, trm)

        t0 = in_ref[0, 2]
        t1 = in_ref[0, 3]
        ce_tcl = _ce2(t0, t1, tclm)

        sn = in_ref[0, 4]
        cs = in_ref[0, 5]
        scale = jax.lax.rsqrt(sn * sn + cs * cs + 0.0001)
        snp = sn * scale
        csp = cs * scale

        tp = in_ref[0, 6]
        bt = in_ref[0, 7]
        topm = rad_ref[0, 0]
        botm = rad_ref[0, 1]
        rad_l = _sl1(tp / (topm + 0.01)) + _sl1(bt / (botm + 0.01))

        tcl_sel = tclm == 1
        s0 = posf
        s1 = negf
        s2 = posf * ce_tr
        s3 = negf * ce_tr
        s4 = posf * ce_tcl
        s5 = tclm.astype(jnp.float32)
        s6 = (tnm * tclm).astype(jnp.float32)
        s7 = jnp.where(tcl_sel, rad_l, 0.0)
        s8 = jnp.where(tcl_sel, _sl1_custom(snp, sin_ref[0]), 0.0)
        s9 = jnp.where(tcl_sel, _sl1_custom(csp, cos_ref[0]), 0.0)

        for q, s in enumerate((s0, s1, s2, s3, s4, s5, s6, s7, s8, s9)):
            acc_ref[q * 8:(q + 1) * 8, :] += jnp.sum(
                s.reshape(rb // 8, 8, LANES), axis=0)

        ce_ref[pl.ds(step * rb, rb), :] = jnp.where(negf > 0.0, ce_tr, -1.0)

        @pl.when(step == nb * nblk - 1)
        def _fin():
            (n_pos, n_neg_avail, loss_pos, neg_sum, tcl_sum, n_tcl_sel,
             n_tcl_train, radii_sum, sin_sum, cos_sum) = [
                jnp.sum(acc_ref[q * 8:(q + 1) * 8, :]) for q in range(10)]

            n_neg = jnp.where(
                n_pos > 0.0,
                jnp.minimum(n_neg_avail, jnp.floor(3.0 * n_pos)),
                jnp.float32(100.0))

            cr = min(512, rows_total)
            nch = rows_total // cr

            def _count_ge(cand):
                def it(c, acc):
                    blk = ce_ref[pl.ds(c * cr, cr), :]
                    bits = jax.lax.bitcast_convert_type(blk, jnp.int32)
                    return acc + jnp.sum((bits >= cand).astype(jnp.float32))
                return jax.lax.fori_loop(0, nch, it, jnp.float32(0.0))

            def _topk_rare(_):
                k_f = n_neg

                def bit_it(i, t):
                    cand = t | (jnp.int32(1) << (30 - i))
                    cnt = _count_ge(cand)
                    return jnp.where(cnt >= k_f, cand, t)

                t = jax.lax.fori_loop(0, 31, bit_it, jnp.int32(0))

                def fin_it(c, carry):
                    s, n = carry
                    blk = ce_ref[pl.ds(c * cr, cr), :]
                    bits = jax.lax.bitcast_convert_type(blk, jnp.int32)
                    gt = bits > t
                    s = s + jnp.sum(jnp.where(gt, blk, 0.0))
                    n = n + jnp.sum(gt.astype(jnp.float32))
                    return (s, n)

                s_gt, n_gt = jax.lax.fori_loop(
                    0, nch, fin_it, (jnp.float32(0.0), jnp.float32(0.0)))
                t_val = jax.lax.bitcast_convert_type(t, jnp.float32)
                return s_gt + (k_f - n_gt) * t_val

            topk_sum = jax.lax.cond(
                n_neg >= n_neg_avail, lambda _: neg_sum, _topk_rare, 0)

            loss_tr = (loss_pos + topk_sum) / (n_pos + n_neg)
            loss_tcl = jnp.where(
                n_pos > 0.0, tcl_sum / jnp.maximum(n_pos, 1.0), 0.0)
            denom = jnp.maximum(n_tcl_sel, 1.0)
            cond = n_tcl_train > 0.0
            loss_radii = jnp.where(cond, radii_sum / denom, 0.0)
            loss_sin = jnp.where(cond, sin_sum / denom, 0.0)
            loss_cos = jnp.where(cond, cos_sum / denom, 0.0)

            gce = _ce2(gp_ref[0], gp_ref[1], gl_ref[...])
            gcn_loss = jnp.sum(gce) / jnp.float32(g_rows * LANES)

            lane = jax.lax.broadcasted_iota(jnp.int32, (1, LANES), 1)
            outv = (jnp.where(lane == 0, loss_tr, 0.0)
                    + jnp.where(lane == 1, loss_tcl, 0.0)
                    + jnp.where(lane == 2, loss_sin, 0.0)
                    + jnp.where(lane == 3, loss_cos, 0.0)
                    + jnp.where(lane == 4, loss_radii, 0.0)
                    + jnp.where(lane == 5, gcn_loss, 0.0))
            out_ref[...] = outv

    return body


def _run(inputs, gcn_pred, gcn_labels, train_mask, tr_mask, tcl_mask,
         radii_map, sin_map, cos_map, interpret=False):
    b, c, h, w = inputs.shape
    pix = h * w
    rb = min(512, pix // LANES)
    nblk = pix // (rb * LANES)
    rows_b = pix // LANES
    rows_total = b * rows_b
    g = gcn_pred.shape[0]
    g_rows = g // LANES

    x = inputs.reshape(b, 8, rows_b, LANES)
    trm = tr_mask.astype(jnp.int32).reshape(b, rows_b, LANES)
    tnm = train_mask.astype(jnp.int32).reshape(b, rows_b, LANES)
    tcl = tcl_mask.astype(jnp.int32).reshape(b, rows_b, LANES)
    rad = jnp.transpose(radii_map.reshape(b, pix, 2), (0, 2, 1))
    rad = rad.reshape(b, 2, rows_b, LANES)
    snm = sin_map.reshape(b, rows_b, LANES)
    csm = cos_map.reshape(b, rows_b, LANES)
    gp = jnp.transpose(gcn_pred, (1, 0)).reshape(2, g_rows, LANES)
    gl = gcn_labels.astype(jnp.int32).reshape(g_rows, LANES)

    body = _make_body(nblk, rb, rows_total, g_rows)
    out = pl.pallas_call(
        body,
        grid=(b, nblk),
        in_specs=[
            pl.BlockSpec((1, 8, rb, LANES), lambda i, j: (i, 0, j, 0)),
            pl.BlockSpec((1, rb, LANES), lambda i, j: (i, j, 0)),
            pl.BlockSpec((1, rb, LANES), lambda i, j: (i, j, 0)),
            pl.BlockSpec((1, rb, LANES), lambda i, j: (i, j, 0)),
            pl.BlockSpec((1, 2, rb, LANES), lambda i, j: (i, 0, j, 0)),
            pl.BlockSpec((1, rb, LANES), lambda i, j: (i, j, 0)),
            pl.BlockSpec((1, rb, LANES), lambda i, j: (i, j, 0)),
            pl.BlockSpec((2, g_rows, LANES), lambda i, j: (0, 0, 0)),
            pl.BlockSpec((g_rows, LANES), lambda i, j: (0, 0)),
        ],
        out_specs=pl.BlockSpec((1, LANES), lambda i, j: (0, 0)),
        out_shape=jax.ShapeDtypeStruct((1, LANES), jnp.float32),
        scratch_shapes=[
            pltpu.VMEM((80, LANES), jnp.float32),
            pltpu.VMEM((rows_total, LANES), jnp.float32),
        ],
        interpret=interpret,
    )(x, trm, tnm, tcl, rad, snm, csm, gp, gl)
    return (out[0, 0], out[0, 1], out[0, 2], out[0, 3], out[0, 4], out[0, 5])


def kernel(inputs, gcn_pred, gcn_labels, train_mask, tr_mask, tcl_mask,
           radii_map, sin_map, cos_map):
    return _run(inputs, gcn_pred, gcn_labels, train_mask, tr_mask, tcl_mask,
                radii_map, sin_map, cos_map)
